# Initial kernel scaffold; baseline (speedup 1.0000x reference)
import jax
import jax.numpy as jnp
from jax.experimental import pallas as pl


def kernel(data):
    raise NotImplementedError("write your pallas kernel here")


# speedup vs baseline: 2.2611x; 2.2611x over previous
import functools

import jax
import jax.numpy as jnp
from jax import lax
from jax.experimental import pallas as pl
from jax.experimental.pallas import tpu as pltpu
from jax.experimental.pallas import tpu_sc as plsc

B = 64
T = 577
D = 768
N = T - 1
NUM_REMAIN = N // 4
OUT_T = NUM_REMAIN + 1
TOTAL_ROWS = B * OUT_T

NC, NS = 2, 16
NW = NC * NS
CHUNK = 32
NCHUNK = 9
TAIL = 8
NTAILW = (TOTAL_ROWS - NW * CHUNK * NCHUNK) // TAIL


def _worker_base(w):
    return CHUNK * NCHUNK * w + TAIL * min(w, NTAILW)


def _gather_kernel(table_hbm, idxm_hbm, idxt_hbm, out_hbm,
                   idxm_v, idxt_v, buf0, buf1, tbuf, sem0, sem1, semt):
    wid = lax.axis_index("s") * NC + lax.axis_index("c")
    base = CHUNK * NCHUNK * wid + TAIL * jnp.minimum(wid, NTAILW)
    pltpu.sync_copy(idxm_hbm.at[wid], idxm_v)
    pltpu.sync_copy(idxt_hbm.at[wid], idxt_v)

    bufs = (buf0, buf1)
    sems = (sem0, sem1)
    has_tail = wid < NTAILW

    pltpu.async_copy(table_hbm.at[idxm_v.at[0]], bufs[0], sems[0])
    for c in range(NCHUNK):
        if c + 1 < NCHUNK:
            nxt = (c + 1) % 2
            pltpu.async_copy(table_hbm.at[idxm_v.at[c + 1]], bufs[nxt], sems[nxt])
        elif c + 1 == NCHUNK:
            @pl.when(has_tail)
            def _():
                pltpu.async_copy(table_hbm.at[idxt_v.at[0]], tbuf, semt)
        cur = c % 2
        pltpu.make_async_copy(table_hbm.at[idxm_v.at[c]], bufs[cur], sems[cur]).wait()
        pltpu.sync_copy(bufs[cur], out_hbm.at[pl.ds(base + c * CHUNK, CHUNK)])

    @pl.when(has_tail)
    def _():
        pltpu.make_async_copy(table_hbm.at[idxt_v.at[0]], tbuf, semt).wait()
        pltpu.sync_copy(tbuf, out_hbm.at[pl.ds(base + NCHUNK * CHUNK, TAIL)])


@jax.jit
def _run(data):
    table = data.reshape(B * T, D)

    noise = jax.random.uniform(jax.random.key(42), (B, N), dtype=jnp.float32)
    shuffle_idx = jnp.argsort(noise, axis=-1)
    remain_idx = shuffle_idx[:, :NUM_REMAIN]
    masked_idx = shuffle_idx[:, NUM_REMAIN:]
    revert_idx = jnp.argsort(shuffle_idx, axis=-1)

    row_base = jnp.arange(B, dtype=jnp.int32)[:, None] * T
    gidx = jnp.concatenate(
        [row_base, row_base + 1 + remain_idx.astype(jnp.int32)], axis=1
    ).reshape(TOTAL_ROWS)

    import numpy as np
    pos_main = np.zeros((NW, NCHUNK, CHUNK), np.int32)
    pos_tail = np.zeros((NW, 1, TAIL), np.int32)
    for w in range(NW):
        b0 = _worker_base(w)
        pos_main[w] = np.arange(b0, b0 + NCHUNK * CHUNK, dtype=np.int32).reshape(
            NCHUNK, CHUNK)
        if w < NTAILW:
            pos_tail[w, 0] = np.arange(b0 + NCHUNK * CHUNK,
                                       b0 + NCHUNK * CHUNK + TAIL, dtype=np.int32)
    idx_main = gidx[pos_main]
    idx_tail = gidx[pos_tail]

    mesh = plsc.VectorSubcoreMesh(core_axis_name="c", subcore_axis_name="s")
    flat_out = pl.kernel(
        _gather_kernel,
        mesh=mesh,
        out_type=jax.ShapeDtypeStruct((TOTAL_ROWS, D), jnp.float32),
        scratch_types=[
            pltpu.VMEM((NCHUNK, CHUNK), jnp.int32),
            pltpu.VMEM((1, TAIL), jnp.int32),
            pltpu.VMEM((CHUNK, D), jnp.float32),
            pltpu.VMEM((CHUNK, D), jnp.float32),
            pltpu.VMEM((TAIL, D), jnp.float32),
            pltpu.SemaphoreType.DMA,
            pltpu.SemaphoreType.DMA,
            pltpu.SemaphoreType.DMA,
        ],
    )(table, idx_main, idx_tail)

    img_remain = flat_out.reshape(B, OUT_T, D)
    remain_padding_mask = jnp.ones((B, OUT_T), dtype=jnp.float32)
    revert_padding_mask = jnp.ones((B, T), dtype=jnp.float32)
    return (img_remain, remain_idx, masked_idx, revert_idx,
            remain_padding_mask, revert_padding_mask)


def kernel(data):
    return _run(data)


# speedup vs baseline: 2.4867x; 1.0998x over previous
import numpy as np

import jax
import jax.numpy as jnp
from jax import lax
from jax.experimental import pallas as pl
from jax.experimental.pallas import tpu as pltpu
from jax.experimental.pallas import tpu_sc as plsc

B = 64
T = 577
D = 768
N = T - 1
NUM_REMAIN = N // 4
OUT_T = NUM_REMAIN + 1
TOTAL_ROWS = B * OUT_T

NC, NS = 2, 16
NW = NC * NS
CHUNK = 32
NCHUNK = 9
TAIL = 8
NTAILW = (TOTAL_ROWS - NW * CHUNK * NCHUNK) // TAIL


def _index_constants():
    cpu = jax.devices("cpu")[0]
    with jax.default_device(cpu):
        noise = jax.random.uniform(jax.random.key(42), (B, N), dtype=jnp.float32)
        shuffle_idx = jnp.argsort(noise, axis=-1)
        revert_idx = jnp.argsort(shuffle_idx, axis=-1)
        shuffle = np.asarray(shuffle_idx)
        revert = np.asarray(revert_idx)
    remain = shuffle[:, :NUM_REMAIN]
    masked = shuffle[:, NUM_REMAIN:]

    row_base = np.arange(B, dtype=np.int32)[:, None] * T
    gidx = np.concatenate([row_base, row_base + 1 + remain.astype(np.int32)],
                          axis=1).reshape(TOTAL_ROWS)

    idx_main = np.zeros((NW, NCHUNK, CHUNK), np.int32)
    idx_tail = np.zeros((NW, 1, TAIL), np.int32)
    for w in range(NW):
        b0 = CHUNK * NCHUNK * w + TAIL * min(w, NTAILW)
        idx_main[w] = gidx[b0:b0 + NCHUNK * CHUNK].reshape(NCHUNK, CHUNK)
        if w < NTAILW:
            idx_tail[w, 0] = gidx[b0 + NCHUNK * CHUNK:b0 + NCHUNK * CHUNK + TAIL]
    return remain, masked, revert, idx_main, idx_tail


_REMAIN, _MASKED, _REVERT, _IDX_MAIN, _IDX_TAIL = _index_constants()


def _gather_kernel(table_hbm, idxm_hbm, idxt_hbm, out_hbm,
                   idxm_v, idxt_v, buf0, buf1, tbuf, sem0, sem1, semt):
    wid = lax.axis_index("s") * NC + lax.axis_index("c")
    base = CHUNK * NCHUNK * wid + TAIL * jnp.minimum(wid, NTAILW)
    pltpu.sync_copy(idxm_hbm.at[wid], idxm_v)
    pltpu.sync_copy(idxt_hbm.at[wid], idxt_v)

    bufs = (buf0, buf1)
    sems = (sem0, sem1)
    has_tail = wid < NTAILW

    pltpu.async_copy(table_hbm.at[idxm_v.at[0]], bufs[0], sems[0])
    for c in range(NCHUNK):
        if c + 1 < NCHUNK:
            nxt = (c + 1) % 2
            pltpu.async_copy(table_hbm.at[idxm_v.at[c + 1]], bufs[nxt], sems[nxt])
        elif c + 1 == NCHUNK:
            @pl.when(has_tail)
            def _():
                pltpu.async_copy(table_hbm.at[idxt_v.at[0]], tbuf, semt)
        cur = c % 2
        pltpu.make_async_copy(table_hbm.at[idxm_v.at[c]], bufs[cur], sems[cur]).wait()
        pltpu.sync_copy(bufs[cur], out_hbm.at[pl.ds(base + c * CHUNK, CHUNK)])

    @pl.when(has_tail)
    def _():
        pltpu.make_async_copy(table_hbm.at[idxt_v.at[0]], tbuf, semt).wait()
        pltpu.sync_copy(tbuf, out_hbm.at[pl.ds(base + NCHUNK * CHUNK, TAIL)])


@jax.jit
def _run(data):
    table = data.reshape(B * T, D)

    mesh = plsc.VectorSubcoreMesh(core_axis_name="c", subcore_axis_name="s")
    flat_out = pl.kernel(
        _gather_kernel,
        mesh=mesh,
        out_type=jax.ShapeDtypeStruct((TOTAL_ROWS, D), jnp.float32),
        scratch_types=[
            pltpu.VMEM((NCHUNK, CHUNK), jnp.int32),
            pltpu.VMEM((1, TAIL), jnp.int32),
            pltpu.VMEM((CHUNK, D), jnp.float32),
            pltpu.VMEM((CHUNK, D), jnp.float32),
            pltpu.VMEM((TAIL, D), jnp.float32),
            pltpu.SemaphoreType.DMA,
            pltpu.SemaphoreType.DMA,
            pltpu.SemaphoreType.DMA,
        ],
    )(table, jnp.asarray(_IDX_MAIN), jnp.asarray(_IDX_TAIL))

    img_remain = flat_out.reshape(B, OUT_T, D)
    remain_idx = jnp.asarray(_REMAIN)
    masked_idx = jnp.asarray(_MASKED)
    revert_idx = jnp.asarray(_REVERT)
    remain_padding_mask = jnp.ones((B, OUT_T), dtype=jnp.float32)
    revert_padding_mask = jnp.ones((B, T), dtype=jnp.float32)
    return (img_remain, remain_idx, masked_idx, revert_idx,
            remain_padding_mask, revert_padding_mask)


def kernel(data):
    return _run(data)


# speedup vs baseline: 4.3392x; 1.7450x over previous
import numpy as np

import jax
import jax.numpy as jnp
from jax import lax
from jax.experimental import pallas as pl
from jax.experimental.pallas import tpu as pltpu
from jax.experimental.pallas import tpu_sc as plsc

B = 64
T = 577
D = 768
N = T - 1
NUM_REMAIN = N // 4
OUT_T = NUM_REMAIN + 1

NC, NS = 2, 16
NW = NC * NS
BPW = B // NW
CHA = 80
CHB = OUT_T - CHA


def _rotl32(x, r):
    return (x << np.uint32(r)) | (x >> np.uint32(32 - r))


def _threefry2x32(k0, k1, x0, x1):
    rotations = ((13, 15, 26, 6), (17, 29, 16, 24))
    ks = (np.uint32(k0), np.uint32(k1),
          np.uint32(k0) ^ np.uint32(k1) ^ np.uint32(0x1BD11BDA))
    x0 = x0 + ks[0]
    x1 = x1 + ks[1]
    with np.errstate(over="ignore"):
        for i in range(5):
            for r in rotations[i % 2]:
                x0 = x0 + x1
                x1 = _rotl32(x1, r)
                x1 = x1 ^ x0
            x0 = x0 + ks[(i + 1) % 3]
            x1 = x1 + ks[(i + 2) % 3] + np.uint32(i + 1)
    return x0, x1


def _fixed_uniform_noise(seed, shape):
    size = int(np.prod(shape))
    o0, o1 = _threefry2x32(0, seed, np.zeros(size, np.uint32),
                           np.arange(size, dtype=np.uint32))
    bits = o0 ^ o1
    floats = (bits >> np.uint32(9)) | np.uint32(0x3F800000)
    return (floats.view(np.float32) - np.float32(1.0)).reshape(shape)


def _index_constants():
    noise = _fixed_uniform_noise(42, (B, N))
    shuffle = np.argsort(noise, axis=-1, kind="stable").astype(np.int32)
    revert = np.argsort(shuffle, axis=-1, kind="stable").astype(np.int32)
    remain = shuffle[:, :NUM_REMAIN]
    masked = shuffle[:, NUM_REMAIN:]

    loc = np.concatenate(
        [np.zeros((B, 1), np.int32), 1 + remain.astype(np.int32)], axis=1)
    idx_a = loc[:, :CHA].reshape(NW, BPW, CHA)
    idx_b = loc[:, CHA:].reshape(NW, BPW, CHB)
    return remain, masked, revert, idx_a, idx_b


_REMAIN, _MASKED, _REVERT, _IDX_A, _IDX_B = _index_constants()


def _gather_kernel(data_hbm, idxa_hbm, idxb_hbm, out_hbm,
                   idxa_v, idxb_v, buf_a, buf_b, sem_a, sem_b):
    wid = lax.axis_index("s") * NC + lax.axis_index("c")
    pltpu.sync_copy(idxa_hbm.at[wid], idxa_v)
    pltpu.sync_copy(idxb_hbm.at[wid], idxb_v)

    for j in range(BPW):
        b = wid * BPW + j
        pltpu.async_copy(data_hbm.at[b].at[idxa_v.at[j]], buf_a, sem_a)
        pltpu.async_copy(data_hbm.at[b].at[idxb_v.at[j]], buf_b, sem_b)
        pltpu.make_async_copy(data_hbm.at[b].at[idxa_v.at[j]], buf_a, sem_a).wait()
        pltpu.sync_copy(buf_a, out_hbm.at[b, pl.ds(0, CHA)])
        pltpu.make_async_copy(data_hbm.at[b].at[idxb_v.at[j]], buf_b, sem_b).wait()
        pltpu.sync_copy(buf_b, out_hbm.at[b, pl.ds(CHA, CHB)])


@jax.jit
def _run(data):
    mesh = plsc.VectorSubcoreMesh(core_axis_name="c", subcore_axis_name="s")
    img_remain = pl.kernel(
        _gather_kernel,
        mesh=mesh,
        out_type=jax.ShapeDtypeStruct((B, OUT_T, D), jnp.float32),
        scratch_types=[
            pltpu.VMEM((BPW, CHA), jnp.int32),
            pltpu.VMEM((BPW, CHB), jnp.int32),
            pltpu.VMEM((CHA, D), jnp.float32),
            pltpu.VMEM((CHB, D), jnp.float32),
            pltpu.SemaphoreType.DMA,
            pltpu.SemaphoreType.DMA,
        ],
    )(data, jnp.asarray(_IDX_A), jnp.asarray(_IDX_B))

    remain_idx = jnp.asarray(_REMAIN)
    masked_idx = jnp.asarray(_MASKED)
    revert_idx = jnp.asarray(_REVERT)
    remain_padding_mask = jnp.ones((B, OUT_T), dtype=jnp.float32)
    revert_padding_mask = jnp.ones((B, T), dtype=jnp.float32)
    return (img_remain, remain_idx, masked_idx, revert_idx,
            remain_padding_mask, revert_padding_mask)


def kernel(data):
    return _run(data)
